# baseline (speedup 1.0000x reference)
import functools

import jax
import jax.numpy as jnp
from jax import lax
from jax.experimental import pallas as pl
from jax.experimental.pallas import tpu as pltpu
from jax.experimental.pallas import tpu_sc as plsc

B, S, D = 4, 2048, 768
L = 16
NC, NS = 2, 16
NW = NC * NS
SW = S // NW
CH = 32
CPB = SW // CH
NCH = B * CPB
NBUF = 3
AHEAD = NBUF - 1


def _body(x_hbm, tok_hbm, pos_hbm, out_hbm, idx_v, pos_v, buf_v,
          isem, psem, *bufsems):
    gsem = bufsems[:NBUF]
    osem = bufsems[NBUF:]
    wid = lax.axis_index("s") * NC + lax.axis_index("c")
    s0 = wid * SW

    idx_cp = []
    for b in range(B):
        idx_cp.append(pltpu.async_copy(
            x_hbm.at[pl.ds(b * S + s0, SW)],
            idx_v.at[pl.ds(b * SW, SW)], isem))
    pos_cp = pltpu.async_copy(pos_hbm.at[pl.ds(s0, SW)], pos_v, psem)
    for cp in idx_cp:
        cp.wait()

    def gather(i, slot):
        return pltpu.async_copy(
            tok_hbm.at[idx_v.at[pl.ds(i * CH, CH)]],
            buf_v.at[slot], gsem[slot])

    def add_pos(slot, i):
        j = i % CPB

        def row_body(r, carry):
            for c in range(D // L):
                sl = pl.ds(c * L, L)
                plsc.addupdate(buf_v.at[slot, r, sl], pos_v[j * CH + r, sl])
            return carry

        lax.fori_loop(0, CH, row_body, 0)

    g = [None] * NBUF
    o = [None] * NBUF
    for k in range(min(AHEAD, NCH)):
        g[k] = gather(k, k)
    pos_cp.wait()
    for i in range(NCH):
        slot = i % NBUF
        ni = i + AHEAD
        if ni < NCH:
            ns = ni % NBUF
            if o[ns] is not None:
                o[ns].wait()
                o[ns] = None
            g[ns] = gather(ni, ns)
        g[slot].wait()
        add_pos(slot, i)
        b, j = divmod(i, CPB)
        ob = b * S + s0 + j * CH
        o[slot] = pltpu.async_copy(
            buf_v.at[slot], out_hbm.at[pl.ds(ob, CH)], osem[slot])
    for slot in range(NBUF):
        if o[slot] is not None:
            o[slot].wait()


@functools.lru_cache(maxsize=1)
def _build():
    mesh = plsc.VectorSubcoreMesh(core_axis_name="c", subcore_axis_name="s")
    return pl.kernel(
        _body,
        out_type=jax.ShapeDtypeStruct((B * S, D), jnp.float32),
        mesh=mesh,
        scratch_types=[
            pltpu.VMEM((B * SW,), jnp.int32),
            pltpu.VMEM((SW, D), jnp.float32),
            pltpu.VMEM((NBUF, CH, D), jnp.float32),
            pltpu.SemaphoreType.DMA,
            pltpu.SemaphoreType.DMA,
        ] + [pltpu.SemaphoreType.DMA] * (2 * NBUF),
    )


def kernel(x, token_table, pos_table):
    x_flat = jnp.reshape(x, (-1,)).astype(jnp.int32)
    out = _build()(x_flat, token_table, pos_table)
    return jnp.reshape(out, (x.shape[0], x.shape[1], D))


# speedup vs baseline: 1.1772x; 1.1772x over previous
import functools

import jax
import jax.numpy as jnp
from jax import lax
from jax.experimental import pallas as pl
from jax.experimental.pallas import tpu as pltpu
from jax.experimental.pallas import tpu_sc as plsc

B, S, D = 4, 2048, 768
L = 16
NC, NS = 2, 16
NW = NC * NS
SW = S // NW
CH = 16
CPB = SW // CH
NCH = B * CPB
NBUF = 4
AHEAD = NBUF - 1


def _body(x_hbm, tok_hbm, pos_hbm, out_hbm, idx_v, pos_v, buf_v,
          isem, psem, *bufsems):
    gsem = bufsems[:NBUF]
    osem = bufsems[NBUF:]
    wid = lax.axis_index("s") * NC + lax.axis_index("c")
    s0 = wid * SW

    idx_cp = []
    for b in range(B):
        idx_cp.append(pltpu.async_copy(
            x_hbm.at[pl.ds(b * S + s0, SW)],
            idx_v.at[pl.ds(b * SW, SW)], isem))
    pos_cp = pltpu.async_copy(pos_hbm.at[pl.ds(s0, SW)], pos_v, psem)
    for cp in idx_cp:
        cp.wait()

    def gather(i, slot):
        return pltpu.async_copy(
            tok_hbm.at[idx_v.at[pl.ds(i * CH, CH)]],
            buf_v.at[slot], gsem[slot])

    def add_pos(slot, i):
        j = i % CPB

        @plsc.parallel_loop(0, CH)
        def _row(r):
            for c in range(D // L):
                sl = pl.ds(c * L, L)
                plsc.addupdate(buf_v.at[slot, r, sl], pos_v[j * CH + r, sl])

    g = [None] * NBUF
    o = [None] * NBUF
    for k in range(min(AHEAD, NCH)):
        g[k] = gather(k, k)
    pos_cp.wait()
    for i in range(NCH):
        slot = i % NBUF
        ni = i + AHEAD
        if ni < NCH:
            ns = ni % NBUF
            if o[ns] is not None:
                o[ns].wait()
                o[ns] = None
            g[ns] = gather(ni, ns)
        g[slot].wait()
        add_pos(slot, i)
        b, j = divmod(i, CPB)
        ob = b * S + s0 + j * CH
        o[slot] = pltpu.async_copy(
            buf_v.at[slot], out_hbm.at[pl.ds(ob, CH)], osem[slot])
    for slot in range(NBUF):
        if o[slot] is not None:
            o[slot].wait()


@functools.lru_cache(maxsize=1)
def _build():
    mesh = plsc.VectorSubcoreMesh(core_axis_name="c", subcore_axis_name="s")
    return pl.kernel(
        _body,
        out_type=jax.ShapeDtypeStruct((B * S, D), jnp.float32),
        mesh=mesh,
        scratch_types=[
            pltpu.VMEM((B * SW,), jnp.int32),
            pltpu.VMEM((SW, D), jnp.float32),
            pltpu.VMEM((NBUF, CH, D), jnp.float32),
            pltpu.SemaphoreType.DMA,
            pltpu.SemaphoreType.DMA,
        ] + [pltpu.SemaphoreType.DMA] * (2 * NBUF),
    )


def kernel(x, token_table, pos_table):
    x_flat = jnp.reshape(x, (-1,)).astype(jnp.int32)
    out = _build()(x_flat, token_table, pos_table)
    return jnp.reshape(out, (x.shape[0], x.shape[1], D))


# speedup vs baseline: 1.3324x; 1.1318x over previous
import functools

import jax
import jax.numpy as jnp
from jax import lax
from jax.experimental import pallas as pl
from jax.experimental.pallas import tpu as pltpu
from jax.experimental.pallas import tpu_sc as plsc

B, S, D = 4, 2048, 768
L = 16
NC, NS = 2, 16
NW = NC * NS
SW = S // NW
CH = 8
NGRP = SW // CH
NBUF = 2 * B


def _body(x_hbm, tok_hbm, pos_hbm, out_hbm, idx_v, pos_v, buf_v,
          isem, psem, *bufsems):
    gsem = bufsems[:NBUF]
    osem = bufsems[NBUF:]
    wid = lax.axis_index("s") * NC + lax.axis_index("c")
    s0 = wid * SW

    idx_cp = []
    for b in range(B):
        idx_cp.append(pltpu.async_copy(
            x_hbm.at[pl.ds(b * S + s0, SW)],
            idx_v.at[pl.ds(b * SW, SW)], isem))
    pos_cp = pltpu.async_copy(pos_hbm.at[pl.ds(s0, SW)], pos_v, psem)
    for cp in idx_cp:
        cp.wait()

    def gather_group(g, half):
        hs = []
        for b in range(B):
            slot = half * B + b
            hs.append(pltpu.async_copy(
                tok_hbm.at[idx_v.at[pl.ds(b * SW + g * CH, CH)]],
                buf_v.at[slot], gsem[slot]))
        return hs

    def add_pos(g, half):
        @plsc.parallel_loop(0, CH)
        def _row(p):
            for c in range(D // L):
                sl = pl.ds(c * L, L)
                v = pos_v[g * CH + p, sl]
                for b in range(B):
                    plsc.addupdate(buf_v.at[half * B + b, p, sl], v)

    o = [None] * NBUF
    cur = gather_group(0, 0)
    pos_cp.wait()
    for g in range(NGRP):
        half = g % 2
        nxt = None
        if g + 1 < NGRP:
            nhalf = 1 - half
            for b in range(B):
                ns = nhalf * B + b
                if o[ns] is not None:
                    o[ns].wait()
                    o[ns] = None
            nxt = gather_group(g + 1, nhalf)
        for h in cur:
            h.wait()
        add_pos(g, half)
        for b in range(B):
            slot = half * B + b
            ob = b * S + s0 + g * CH
            o[slot] = pltpu.async_copy(
                buf_v.at[slot], out_hbm.at[pl.ds(ob, CH)], osem[slot])
        cur = nxt
    for slot in range(NBUF):
        if o[slot] is not None:
            o[slot].wait()


@functools.lru_cache(maxsize=1)
def _build():
    mesh = plsc.VectorSubcoreMesh(core_axis_name="c", subcore_axis_name="s")
    return pl.kernel(
        _body,
        out_type=jax.ShapeDtypeStruct((B * S, D), jnp.float32),
        mesh=mesh,
        scratch_types=[
            pltpu.VMEM((B * SW,), jnp.int32),
            pltpu.VMEM((SW, D), jnp.float32),
            pltpu.VMEM((NBUF, CH, D), jnp.float32),
            pltpu.SemaphoreType.DMA,
            pltpu.SemaphoreType.DMA,
        ] + [pltpu.SemaphoreType.DMA] * (2 * NBUF),
    )


def kernel(x, token_table, pos_table):
    x_flat = jnp.reshape(x, (-1,)).astype(jnp.int32)
    out = _build()(x_flat, token_table, pos_table)
    return jnp.reshape(out, (x.shape[0], x.shape[1], D))


# speedup vs baseline: 1.3589x; 1.0199x over previous
import functools

import jax
import jax.numpy as jnp
from jax import lax
from jax.experimental import pallas as pl
from jax.experimental.pallas import tpu as pltpu
from jax.experimental.pallas import tpu_sc as plsc

B, S, D = 4, 2048, 768
L = 16
NC, NS = 2, 16
NW = NC * NS
SW = S // NW
CH = 8
NGRP = SW // CH
NGB = 3
NBUF = NGB * B


def _body(x_hbm, tok_hbm, pos_hbm, out_hbm, idx_v, pos_v, buf_v,
          isem, psem, *bufsems):
    gsem = bufsems[:NBUF]
    osem = bufsems[NBUF:]
    wid = lax.axis_index("s") * NC + lax.axis_index("c")
    s0 = wid * SW

    idx_cp = []
    for b in range(B):
        idx_cp.append(pltpu.async_copy(
            x_hbm.at[pl.ds(b * S + s0, SW)],
            idx_v.at[pl.ds(b * SW, SW)], isem))
    pos_cp = pltpu.async_copy(pos_hbm.at[pl.ds(s0, SW)], pos_v, psem)
    for cp in idx_cp:
        cp.wait()

    def gather_group(g, half):
        hs = []
        for b in range(B):
            slot = half * B + b
            hs.append(pltpu.async_copy(
                tok_hbm.at[idx_v.at[pl.ds(b * SW + g * CH, CH)]],
                buf_v.at[slot], gsem[slot]))
        return hs

    def add_pos(g, half):
        @plsc.parallel_loop(0, CH)
        def _row(p):
            for c in range(D // L):
                sl = pl.ds(c * L, L)
                v = pos_v[g * CH + p, sl]
                for b in range(B):
                    plsc.addupdate(buf_v.at[half * B + b, p, sl], v)

    o = [None] * NBUF
    grp = [None] * NGB
    for k in range(NGB - 1):
        grp[k] = gather_group(k, k)
    pos_cp.wait()
    for g in range(NGRP):
        half = g % NGB
        ng = g + NGB - 1
        if ng < NGRP:
            nhalf = ng % NGB
            for b in range(B):
                ns = nhalf * B + b
                if o[ns] is not None:
                    o[ns].wait()
                    o[ns] = None
            grp[nhalf] = gather_group(ng, nhalf)
        for h in grp[half]:
            h.wait()
        add_pos(g, half)
        for b in range(B):
            slot = half * B + b
            ob = b * S + s0 + g * CH
            o[slot] = pltpu.async_copy(
                buf_v.at[slot], out_hbm.at[pl.ds(ob, CH)], osem[slot])
    for slot in range(NBUF):
        if o[slot] is not None:
            o[slot].wait()


@functools.lru_cache(maxsize=1)
def _build():
    mesh = plsc.VectorSubcoreMesh(core_axis_name="c", subcore_axis_name="s")
    return pl.kernel(
        _body,
        out_type=jax.ShapeDtypeStruct((B * S, D), jnp.float32),
        mesh=mesh,
        scratch_types=[
            pltpu.VMEM((B * SW,), jnp.int32),
            pltpu.VMEM((SW, D), jnp.float32),
            pltpu.VMEM((NBUF, CH, D), jnp.float32),
            pltpu.SemaphoreType.DMA,
            pltpu.SemaphoreType.DMA,
        ] + [pltpu.SemaphoreType.DMA] * (2 * NBUF),
    )


def kernel(x, token_table, pos_table):
    x_flat = jnp.reshape(x, (-1,)).astype(jnp.int32)
    out = _build()(x_flat, token_table, pos_table)
    return jnp.reshape(out, (x.shape[0], x.shape[1], D))
